# baseline (speedup 1.0000x reference)
import numpy as np
import jax
import jax.numpy as jnp
from jax import lax
from jax.experimental import pallas as pl
from jax.experimental.pallas import tpu as pltpu


def _blend_weights(o_idx, s_idx, n):
    src = (o_idx + 0.5) * 0.5 - 0.5
    src = jnp.clip(src, 0.0, float(n - 1))
    return jnp.maximum(0.0, 1.0 - jnp.abs(src - s_idx))


def _iota2(shape, dim):
    return lax.broadcasted_iota(jnp.int32, shape, dim).astype(jnp.float32)


def _up2x_nchw(x, aw, ah):
    bb, H, W = x.shape
    y = lax.dot_general(
        x.reshape(bb * H, W), aw, (((1,), (0,)), ((), ())),
        preferred_element_type=jnp.float32,
    ).reshape(bb, H, 2 * W)
    z = lax.dot_general(
        jnp.broadcast_to(ah, (bb,) + ah.shape), y,
        (((2,), (1,)), ((0,), (0,))),
        preferred_element_type=jnp.float32,
    )
    return z


def _up2x_nhwc(x, mk):
    H, W, C = x.shape
    z = lax.dot_general(
        mk, x.reshape(H * W, C), (((1,), (0,)), ((), ())),
        preferred_element_type=jnp.float32,
    )
    return z.reshape(2 * H, 2 * W, C)


def _make_body(H0, W0, H1, W1, nb):
    def body(e0_ref, d0_ref, e1_ref, d1_ref,
             ue0_ref, ud0_ref, ue1_ref, ud1_ref,
             s1_ref, s2_ref, sem_ref):
        C = e0_ref.shape[1]
        i = pl.program_id(0)
        cp1 = pltpu.make_async_copy(
            e1_ref.at[pl.ds(i * nb, nb)], s1_ref, sem_ref.at[0])
        cp2 = pltpu.make_async_copy(
            d1_ref.at[pl.ds(i * nb, nb)], s2_ref, sem_ref.at[1])
        cp1.start()
        cp2.start()

        aw0 = _blend_weights(_iota2((W0, 2 * W0), 1),
                             _iota2((W0, 2 * W0), 0), W0)
        ah0 = _blend_weights(_iota2((2 * H0, H0), 0),
                             _iota2((2 * H0, H0), 1), H0)
        z = _up2x_nchw(e0_ref[...].reshape(nb * C, H0, W0), aw0, ah0)
        ue0_ref[...] = z.reshape(ue0_ref.shape).astype(ue0_ref.dtype)
        z = _up2x_nchw(d0_ref[...].reshape(nb * C, H0, W0), aw0, ah0)
        ud0_ref[...] = z.reshape(ud0_ref.shape).astype(ud0_ref.dtype)

        mshape = (4 * H1 * W1, H1 * W1)
        r = lax.broadcasted_iota(jnp.int32, mshape, 0)
        c = lax.broadcasted_iota(jnp.int32, mshape, 1)
        mk1 = (_blend_weights((r // (2 * W1)).astype(jnp.float32),
                              (c // W1).astype(jnp.float32), H1)
               * _blend_weights((r % (2 * W1)).astype(jnp.float32),
                                (c % W1).astype(jnp.float32), W1))
        mk1b = jnp.broadcast_to(mk1, (nb,) + mk1.shape)
        cp1.wait()
        cp2.wait()
        for x_ref, o_ref in ((s1_ref, ue1_ref), (s2_ref, ud1_ref)):
            xf = x_ref[...].reshape(nb, H1 * W1, x_ref.shape[-1])
            z = lax.dot_general(
                mk1b, xf, (((2,), (1,)), ((0,), (0,))),
                preferred_element_type=jnp.float32,
            )
            o_ref[...] = z.reshape(o_ref.shape).astype(o_ref.dtype)
    return body


def kernel(e0, e1, d0, d1):
    N, C, H0, W0 = e0.shape
    _, _, H1, W1 = e1.shape

    nb = 2 if N % 2 == 0 else 1
    grid = (N // nb,)

    def slab(i):
        return (i, 0, 0, 0)

    e1t = jnp.transpose(e1, (0, 2, 3, 1))
    d1t = jnp.transpose(d1, (0, 2, 3, 1))

    itemsize = jnp.dtype(e0.dtype).itemsize
    out = pl.pallas_call(
        _make_body(H0, W0, H1, W1, nb),
        grid=grid,
        in_specs=[
            pl.BlockSpec((nb, C, H0, W0), slab),
            pl.BlockSpec((nb, C, H0, W0), slab),
            pl.BlockSpec(memory_space=pl.ANY),
            pl.BlockSpec(memory_space=pl.ANY),
        ],
        scratch_shapes=[
            pltpu.MemorySpace.VMEM((nb, H1, W1, C), e1.dtype),
            pltpu.MemorySpace.VMEM((nb, H1, W1, C), d1.dtype),
            pltpu.SemaphoreType.DMA((2,)),
        ],
        out_specs=[
            pl.BlockSpec((nb, C, 2 * H0, 2 * W0), slab),
            pl.BlockSpec((nb, C, 2 * H0, 2 * W0), slab),
            pl.BlockSpec((nb, 2 * H1, 2 * W1, C), slab),
            pl.BlockSpec((nb, 2 * H1, 2 * W1, C), slab),
        ],
        out_shape=[
            jax.ShapeDtypeStruct((N, C, 2 * H0, 2 * W0), e0.dtype),
            jax.ShapeDtypeStruct((N, C, 2 * H0, 2 * W0), d0.dtype),
            jax.ShapeDtypeStruct((N, 2 * H1, 2 * W1, C), e1.dtype),
            jax.ShapeDtypeStruct((N, 2 * H1, 2 * W1, C), d1.dtype),
        ],
        compiler_params=pltpu.CompilerParams(
            dimension_semantics=("parallel",),
            vmem_limit_bytes=60 * 1024 * 1024,
        ),
        cost_estimate=pl.CostEstimate(
            flops=2 * 2 * N * C * (H0 * W0 * 2 * W0 + 2 * H0 * H0 * 2 * W0)
            + 2 * 2 * N * 4 * H1 * W1 * H1 * W1 * C,
            transcendentals=0,
            bytes_accessed=5 * 2 * N * C * (H0 * W0 + H1 * W1) * itemsize,
        ),
    )(e0, d0, e1t, d1t)

    ue0, ud0, ue1t, ud1t = out
    ue1 = jnp.transpose(ue1t, (0, 3, 1, 2))
    ud1 = jnp.transpose(ud1t, (0, 3, 1, 2))
    return ([ue0, ue1], [ud0, ud1])


# speedup vs baseline: 1.0601x; 1.0601x over previous
import numpy as np
import jax
import jax.numpy as jnp
from jax import lax
from jax.experimental import pallas as pl
from jax.experimental.pallas import tpu as pltpu


def _blend_weights(o_idx, s_idx, n):
    src = (o_idx + 0.5) * 0.5 - 0.5
    src = jnp.clip(src, 0.0, float(n - 1))
    return jnp.maximum(0.0, 1.0 - jnp.abs(src - s_idx))


def _iota2(shape, dim):
    return lax.broadcasted_iota(jnp.int32, shape, dim).astype(jnp.float32)


def _up2x_nchw(x, aw, ah):
    bb, H, W = x.shape
    y = lax.dot_general(
        x.reshape(bb * H, W), aw, (((1,), (0,)), ((), ())),
        preferred_element_type=jnp.float32,
    ).reshape(bb, H, 2 * W)
    z = lax.dot_general(
        jnp.broadcast_to(ah, (bb,) + ah.shape), y,
        (((2,), (1,)), ((0,), (0,))),
        preferred_element_type=jnp.float32,
    )
    return z


def _up2x_nhwc(x, mk):
    H, W, C = x.shape
    z = lax.dot_general(
        mk, x.reshape(H * W, C), (((1,), (0,)), ((), ())),
        preferred_element_type=jnp.float32,
    )
    return z.reshape(2 * H, 2 * W, C)


def _make_body(H0, W0, H1, W1):
    def body(e0_ref, d0_ref, e1_ref, d1_ref,
             ue0_ref, ud0_ref, ue1_ref, ud1_ref):
        nb, C = e0_ref.shape[:2]
        mshape = (4 * H1 * W1, H1 * W1)
        r = lax.broadcasted_iota(jnp.int32, mshape, 0)
        c = lax.broadcasted_iota(jnp.int32, mshape, 1)
        mk1 = (_blend_weights((r // (2 * W1)).astype(jnp.float32),
                              (c // W1).astype(jnp.float32), H1)
               * _blend_weights((r % (2 * W1)).astype(jnp.float32),
                                (c % W1).astype(jnp.float32), W1))
        mk1b = jnp.broadcast_to(mk1, (nb,) + mk1.shape)
        for x_ref, o_ref in ((e1_ref, ue1_ref), (d1_ref, ud1_ref)):
            xf = x_ref[...].reshape(nb, H1 * W1, x_ref.shape[-1])
            z = lax.dot_general(
                mk1b, xf, (((2,), (1,)), ((0,), (0,))),
                preferred_element_type=jnp.float32,
            )
            o_ref[...] = z.reshape(o_ref.shape).astype(o_ref.dtype)

        aw0 = _blend_weights(_iota2((W0, 2 * W0), 1),
                             _iota2((W0, 2 * W0), 0), W0)
        ah0 = _blend_weights(_iota2((2 * H0, H0), 0),
                             _iota2((2 * H0, H0), 1), H0)
        z = _up2x_nchw(e0_ref[...].reshape(nb * C, H0, W0), aw0, ah0)
        ue0_ref[...] = z.reshape(ue0_ref.shape).astype(ue0_ref.dtype)
        z = _up2x_nchw(d0_ref[...].reshape(nb * C, H0, W0), aw0, ah0)
        ud0_ref[...] = z.reshape(ud0_ref.shape).astype(ud0_ref.dtype)
    return body


def kernel(e0, e1, d0, d1):
    N, C, H0, W0 = e0.shape
    _, _, H1, W1 = e1.shape

    nb = 2 if N % 2 == 0 else 1
    grid = (N // nb,)

    def slab(i):
        return (i, 0, 0, 0)

    e1t = jnp.transpose(e1, (0, 2, 3, 1))
    d1t = jnp.transpose(d1, (0, 2, 3, 1))

    itemsize = jnp.dtype(e0.dtype).itemsize
    out = pl.pallas_call(
        _make_body(H0, W0, H1, W1),
        grid=grid,
        in_specs=[
            pl.BlockSpec((nb, C, H0, W0), slab),
            pl.BlockSpec((nb, C, H0, W0), slab),
            pl.BlockSpec((nb, H1, W1, C), slab),
            pl.BlockSpec((nb, H1, W1, C), slab),
        ],
        out_specs=[
            pl.BlockSpec((nb, C, 2 * H0, 2 * W0), slab),
            pl.BlockSpec((nb, C, 2 * H0, 2 * W0), slab),
            pl.BlockSpec((nb, 2 * H1, 2 * W1, C), slab),
            pl.BlockSpec((nb, 2 * H1, 2 * W1, C), slab),
        ],
        out_shape=[
            jax.ShapeDtypeStruct((N, C, 2 * H0, 2 * W0), e0.dtype),
            jax.ShapeDtypeStruct((N, C, 2 * H0, 2 * W0), d0.dtype),
            jax.ShapeDtypeStruct((N, 2 * H1, 2 * W1, C), e1.dtype),
            jax.ShapeDtypeStruct((N, 2 * H1, 2 * W1, C), d1.dtype),
        ],
        compiler_params=pltpu.CompilerParams(
            dimension_semantics=("parallel",),
            vmem_limit_bytes=60 * 1024 * 1024,
        ),
        cost_estimate=pl.CostEstimate(
            flops=2 * 2 * N * C * (H0 * W0 * 2 * W0 + 2 * H0 * H0 * 2 * W0)
            + 2 * 2 * N * 4 * H1 * W1 * H1 * W1 * C,
            transcendentals=0,
            bytes_accessed=5 * 2 * N * C * (H0 * W0 + H1 * W1) * itemsize,
        ),
    )(e0, d0, e1t, d1t)

    ue0, ud0, ue1t, ud1t = out
    ue1 = jnp.transpose(ue1t, (0, 3, 1, 2))
    ud1 = jnp.transpose(ud1t, (0, 3, 1, 2))
    return ([ue0, ue1], [ud0, ud1])


# speedup vs baseline: 1.0602x; 1.0001x over previous
import jax
import jax.numpy as jnp
from jax import lax
from jax.experimental import pallas as pl
from jax.experimental.pallas import tpu as pltpu


def _blend_weights(o_idx, s_idx, n):
    src = (o_idx + 0.5) * 0.5 - 0.5
    src = jnp.clip(src, 0.0, float(n - 1))
    return jnp.maximum(0.0, 1.0 - jnp.abs(src - s_idx))


def _iota2(shape, dim):
    return lax.broadcasted_iota(jnp.int32, shape, dim).astype(jnp.float32)


def _up2x_nchw(x, aw, ah):
    bb, H, W = x.shape
    y = lax.dot_general(
        x.reshape(bb * H, W), aw, (((1,), (0,)), ((), ())),
        preferred_element_type=jnp.float32,
    ).reshape(bb, H, 2 * W)
    z = lax.dot_general(
        jnp.broadcast_to(ah, (bb,) + ah.shape), y,
        (((2,), (1,)), ((0,), (0,))),
        preferred_element_type=jnp.float32,
    )
    return z


def _make_body(H0, W0, H1, W1):
    def body(e0_ref, d0_ref, e1_ref, d1_ref,
             ue0_ref, ud0_ref, ue1_ref, ud1_ref):
        nb, C = e0_ref.shape[:2]
        mshape = (4 * H1 * W1, H1 * W1)
        r = lax.broadcasted_iota(jnp.int32, mshape, 0)
        c = lax.broadcasted_iota(jnp.int32, mshape, 1)
        mk1 = (_blend_weights((r // (2 * W1)).astype(jnp.float32),
                              (c // W1).astype(jnp.float32), H1)
               * _blend_weights((r % (2 * W1)).astype(jnp.float32),
                                (c % W1).astype(jnp.float32), W1))
        mk1b = jnp.broadcast_to(mk1, (nb,) + mk1.shape)
        for x_ref, o_ref in ((e1_ref, ue1_ref), (d1_ref, ud1_ref)):
            xf = x_ref[...].reshape(nb, H1 * W1, x_ref.shape[-1])
            z = lax.dot_general(
                mk1b, xf, (((2,), (1,)), ((0,), (0,))),
                preferred_element_type=jnp.float32,
            )
            o_ref[...] = z.reshape(o_ref.shape).astype(o_ref.dtype)

        aw0 = _blend_weights(_iota2((W0, 2 * W0), 1),
                             _iota2((W0, 2 * W0), 0), W0)
        ah0 = _blend_weights(_iota2((2 * H0, H0), 0),
                             _iota2((2 * H0, H0), 1), H0)
        z = _up2x_nchw(e0_ref[...].reshape(nb * C, H0, W0), aw0, ah0)
        ue0_ref[...] = z.reshape(ue0_ref.shape).astype(ue0_ref.dtype)
        z = _up2x_nchw(d0_ref[...].reshape(nb * C, H0, W0), aw0, ah0)
        ud0_ref[...] = z.reshape(ud0_ref.shape).astype(ud0_ref.dtype)
    return body


def kernel(e0, e1, d0, d1):
    N, C, H0, W0 = e0.shape
    _, _, H1, W1 = e1.shape

    nb = 2 if N % 2 == 0 else 1
    grid = (N // nb,)

    def slab(i):
        return (i, 0, 0, 0)

    e1t = jnp.transpose(e1, (0, 2, 3, 1))
    d1t = jnp.transpose(d1, (0, 2, 3, 1))

    itemsize = jnp.dtype(e0.dtype).itemsize
    out = pl.pallas_call(
        _make_body(H0, W0, H1, W1),
        grid=grid,
        in_specs=[
            pl.BlockSpec((nb, C, H0, W0), slab),
            pl.BlockSpec((nb, C, H0, W0), slab),
            pl.BlockSpec((nb, H1, W1, C), slab),
            pl.BlockSpec((nb, H1, W1, C), slab),
        ],
        out_specs=[
            pl.BlockSpec((nb, C, 2 * H0, 2 * W0), slab),
            pl.BlockSpec((nb, C, 2 * H0, 2 * W0), slab),
            pl.BlockSpec((nb, 2 * H1, 2 * W1, C), slab),
            pl.BlockSpec((nb, 2 * H1, 2 * W1, C), slab),
        ],
        out_shape=[
            jax.ShapeDtypeStruct((N, C, 2 * H0, 2 * W0), e0.dtype),
            jax.ShapeDtypeStruct((N, C, 2 * H0, 2 * W0), d0.dtype),
            jax.ShapeDtypeStruct((N, 2 * H1, 2 * W1, C), e1.dtype),
            jax.ShapeDtypeStruct((N, 2 * H1, 2 * W1, C), d1.dtype),
        ],
        compiler_params=pltpu.CompilerParams(
            dimension_semantics=("parallel",),
            vmem_limit_bytes=60 * 1024 * 1024,
        ),
        cost_estimate=pl.CostEstimate(
            flops=2 * 2 * N * C * (H0 * W0 * 2 * W0 + 2 * H0 * H0 * 2 * W0)
            + 2 * 2 * N * 4 * H1 * W1 * H1 * W1 * C,
            transcendentals=0,
            bytes_accessed=5 * 2 * N * C * (H0 * W0 + H1 * W1) * itemsize,
        ),
    )(e0, d0, e1t, d1t)

    ue0, ud0, ue1t, ud1t = out
    ue1 = jnp.transpose(ue1t, (0, 3, 1, 2))
    ud1 = jnp.transpose(ud1t, (0, 3, 1, 2))
    return ([ue0, ue1], [ud0, ud1])
